# baseline (speedup 1.0000x reference)
import functools

import jax
import jax.numpy as jnp
from jax import lax
from jax.experimental import pallas as pl
from jax.experimental.pallas import tpu as pltpu
from jax.experimental.pallas import tpu_sc as plsc

N = 65536
B = 64
SPN = 33
NNZ = N * SPN
XC = 512
EW = SPN * XC // 2
GROUPS = EW // 16
NCHUNK = N // XC
NW = 32
COLS_PER_W = B // NW
MAGIC = 127101


def _sc_spmm(x_flat, rows_pk, vals_pk, bias_flat):
    mesh = plsc.VectorSubcoreMesh(core_axis_name="c", subcore_axis_name="s")

    @functools.partial(
        pl.kernel,
        out_type=jax.ShapeDtypeStruct((B * N,), jnp.float32),
        mesh=mesh,
        scratch_types=[
            pltpu.VMEM((N,), jnp.float32),
            pltpu.VMEM((EW,), jnp.int32),
            pltpu.VMEM((EW,), jnp.int32),
            pltpu.VMEM((EW,), jnp.int32),
            pltpu.VMEM((EW,), jnp.int32),
            pltpu.VMEM((XC,), jnp.float32),
            pltpu.VMEM((XC,), jnp.float32),
            pltpu.SemaphoreType.DMA,
            pltpu.SemaphoreType.DMA,
        ],
        compiler_params=pltpu.CompilerParams(needs_layout_passes=False),
    )
    def k(x_hbm, rows_hbm, vals_hbm, bias_hbm, out_hbm, acc,
          rb0, rb1, vb0, vb1, xb0, xb1, sem0, sem1):
        wid = lax.axis_index("s") * 2 + lax.axis_index("c")
        lane = lax.iota(jnp.int32, 16)
        rbufs, vbufs, xbufs, sems = (rb0, rb1), (vb0, vb1), (xb0, xb1), \
            (sem0, sem1)

        def issue(c, bi, xoff):
            pltpu.async_copy(rows_hbm.at[pl.ds(c * EW, EW)], rbufs[bi],
                             sems[bi])
            pltpu.async_copy(vals_hbm.at[pl.ds(c * EW, EW)], vbufs[bi],
                             sems[bi])
            pltpu.async_copy(x_hbm.at[pl.ds(xoff + c * XC, XC)],
                             xbufs[bi], sems[bi])

        def drain(bi):
            pltpu.make_async_copy(rows_hbm.at[pl.ds(0, EW)], rbufs[bi],
                                  sems[bi]).wait()
            pltpu.make_async_copy(vals_hbm.at[pl.ds(0, EW)], vbufs[bi],
                                  sems[bi]).wait()
            pltpu.make_async_copy(x_hbm.at[pl.ds(0, XC)], xbufs[bi],
                                  sems[bi]).wait()

        for col in range(COLS_PER_W):
            b = wid * COLS_PER_W + col
            pltpu.sync_copy(bias_hbm, acc)
            xoff = b * N
            issue(0, 0, xoff)
            issue(1, 1, xoff)

            def pair_body(j, _):
                for bi in range(2):
                    cc = j * 2 + bi
                    drain(bi)
                    rbuf, vbuf, xbuf = rbufs[bi], vbufs[bi], xbufs[bi]

                    def group_body(g, _):
                        wbase = g * 16
                        me = (lane + wbase) * (2 * MAGIC)
                        ce = lax.shift_right_logical(me, 22)
                        co = lax.shift_right_logical(me + MAGIC, 22)
                        xe = plsc.load_gather(xbuf, [ce])
                        xo = plsc.load_gather(xbuf, [co])
                        rw = rbuf[pl.ds(wbase, 16)]
                        vw = vbuf[pl.ds(wbase, 16)]
                        r_lo = rw & 0xFFFF
                        r_hi = lax.shift_right_logical(rw, 16)
                        v_lo = plsc.bitcast(lax.shift_left(vw, 16),
                                            jnp.float32)
                        v_hi = plsc.bitcast(vw & jnp.int32(-65536),
                                            jnp.float32)
                        plsc.addupdate_scatter(acc, [r_lo], v_lo * xe)
                        plsc.addupdate_scatter(acc, [r_hi], v_hi * xo)
                        return 0

                    lax.fori_loop(0, GROUPS, group_body, 0)
                    nc = cc + 2

                    @pl.when(nc < NCHUNK)
                    def _():
                        issue(nc, bi, xoff)
                return 0

            lax.fori_loop(0, NCHUNK // 2, pair_body, 0)
            pltpu.sync_copy(acc, out_hbm.at[pl.ds(b * N, N)])

    return k(x_flat, rows_pk, vals_pk, bias_flat)


def kernel(x, indices, values, bias):
    rows_pk = lax.bitcast_convert_type(
        indices[0].astype(jnp.uint16).reshape(NNZ // 2, 2), jnp.int32)
    vals_pk = lax.bitcast_convert_type(
        values.astype(jnp.bfloat16).reshape(NNZ // 2, 2), jnp.int32)
    out_flat = _sc_spmm(x.reshape(-1), rows_pk, vals_pk, bias.reshape(-1))
    return out_flat.reshape(B, N)


# speedup vs baseline: 2.2368x; 2.2368x over previous
import functools

import jax
import jax.numpy as jnp
from jax import lax
from jax.experimental import pallas as pl
from jax.experimental.pallas import tpu as pltpu
from jax.experimental.pallas import tpu_sc as plsc

N = 65536
B = 64
SPN = 33
CHUNK = 512
NCHUNK = N // CHUNK
EW = SPN * CHUNK // 2
NW = 32
COLS_PER_W = B // NW


def _sc_spmm(x_flat, rows_pk, vals_pk, bias_flat):
    mesh = plsc.VectorSubcoreMesh(core_axis_name="c", subcore_axis_name="s")

    @functools.partial(
        pl.kernel,
        out_type=jax.ShapeDtypeStruct((B * N,), jnp.float32),
        mesh=mesh,
        scratch_types=[
            pltpu.VMEM((N,), jnp.float32),
            pltpu.VMEM((EW,), jnp.int32),
            pltpu.VMEM((EW,), jnp.int32),
            pltpu.VMEM((EW,), jnp.int32),
            pltpu.VMEM((EW,), jnp.int32),
            pltpu.VMEM((CHUNK,), jnp.float32),
            pltpu.VMEM((CHUNK,), jnp.float32),
            pltpu.SemaphoreType.DMA,
            pltpu.SemaphoreType.DMA,
        ],
        compiler_params=pltpu.CompilerParams(needs_layout_passes=False),
    )
    def k(x_hbm, rows_hbm, vals_hbm, bias_hbm, out_hbm, acc,
          rb0, rb1, vb0, vb1, xb0, xb1, sem0, sem1):
        wid = lax.axis_index("s") * 2 + lax.axis_index("c")
        lane = lax.iota(jnp.int32, 16)
        ev_idx = lane * 2
        od_idx = ev_idx + 1
        rbufs, vbufs, xbufs, sems = (rb0, rb1), (vb0, vb1), (xb0, xb1), \
            (sem0, sem1)

        def issue(c, bi, xoff):
            pltpu.async_copy(rows_hbm.at[pl.ds(c * EW, EW)], rbufs[bi],
                             sems[bi])
            pltpu.async_copy(vals_hbm.at[pl.ds(c * EW, EW)], vbufs[bi],
                             sems[bi])
            pltpu.async_copy(x_hbm.at[pl.ds(xoff + c * CHUNK, CHUNK)],
                             xbufs[bi], sems[bi])

        def drain(bi):
            pltpu.make_async_copy(rows_hbm.at[pl.ds(0, EW)], rbufs[bi],
                                  sems[bi]).wait()
            pltpu.make_async_copy(vals_hbm.at[pl.ds(0, EW)], vbufs[bi],
                                  sems[bi]).wait()
            pltpu.make_async_copy(x_hbm.at[pl.ds(0, CHUNK)], xbufs[bi],
                                  sems[bi]).wait()

        for col in range(COLS_PER_W):
            b = wid * COLS_PER_W + col
            pltpu.sync_copy(bias_hbm, acc)
            xoff = b * N
            issue(0, 0, xoff)
            issue(1, 1, xoff)

            def pair_body(j, _):
                for bi in range(2):
                    cc = j * 2 + bi
                    drain(bi)
                    rbuf, vbuf, xbuf = rbufs[bi], vbufs[bi], xbufs[bi]

                    @plsc.parallel_loop(0, CHUNK // 32, unroll=2)
                    def group_body(g):
                        gx = g * 32
                        xe = plsc.load_gather(xbuf, [gx + ev_idx])
                        xo = plsc.load_gather(xbuf, [gx + od_idx])
                        g16 = g * 16

                        for s in range(SPN):
                            off = s * (CHUNK // 2) + g16
                            rw = rbuf[pl.ds(off, 16)]
                            vw = vbuf[pl.ds(off, 16)]
                            r_lo = rw & 0xFFFF
                            r_hi = lax.shift_right_logical(rw, 16)
                            v_lo = plsc.bitcast(lax.shift_left(vw, 16),
                                                jnp.float32)
                            v_hi = plsc.bitcast(vw & jnp.int32(-65536),
                                                jnp.float32)
                            plsc.addupdate_scatter(acc, [r_lo], v_lo * xe)
                            plsc.addupdate_scatter(acc, [r_hi], v_hi * xo)
                    nc = cc + 2

                    @pl.when(nc < NCHUNK)
                    def _():
                        issue(nc, bi, xoff)
                return 0

            lax.fori_loop(0, NCHUNK // 2, pair_body, 0)
            pltpu.sync_copy(acc, out_hbm.at[pl.ds(b * N, N)])

    return k(x_flat, rows_pk, vals_pk, bias_flat)


def kernel(x, indices, values, bias):
    rows = indices[0].astype(jnp.uint16)
    rows_b = rows.reshape(NCHUNK, CHUNK, SPN).transpose(0, 2, 1)
    rows_pk = lax.bitcast_convert_type(
        rows_b.reshape(NCHUNK, SPN, CHUNK // 2, 2), jnp.int32).reshape(-1)
    vals_b = values.astype(jnp.bfloat16).reshape(
        NCHUNK, CHUNK, SPN).transpose(0, 2, 1)
    vals_pk = lax.bitcast_convert_type(
        vals_b.reshape(NCHUNK, SPN, CHUNK // 2, 2), jnp.int32).reshape(-1)
    out_flat = _sc_spmm(x.reshape(-1), rows_pk, vals_pk, bias.reshape(-1))
    return out_flat.reshape(B, N)


# speedup vs baseline: 2.3058x; 1.0309x over previous
import functools

import jax
import jax.numpy as jnp
from jax import lax
from jax.experimental import pallas as pl
from jax.experimental.pallas import tpu as pltpu
from jax.experimental.pallas import tpu_sc as plsc

N = 65536
B = 64
SPN = 33
XC = 512
EW = SPN * XC
GROUPS = XC // 16
NCHUNK = N // XC
NW = 32
COLS_PER_W = B // NW


def _sc_spmm(x_flat, edges, bias_flat):
    mesh = plsc.VectorSubcoreMesh(core_axis_name="c", subcore_axis_name="s")

    @functools.partial(
        pl.kernel,
        out_type=jax.ShapeDtypeStruct((B * N,), jnp.float32),
        mesh=mesh,
        scratch_types=[
            pltpu.VMEM((N,), jnp.float32),
            pltpu.VMEM((EW,), jnp.int32),
            pltpu.VMEM((EW,), jnp.int32),
            pltpu.VMEM((XC,), jnp.float32),
            pltpu.VMEM((XC,), jnp.float32),
            pltpu.SemaphoreType.DMA,
            pltpu.SemaphoreType.DMA,
        ],
        compiler_params=pltpu.CompilerParams(needs_layout_passes=False),
    )
    def k(x_hbm, edges_hbm, bias_hbm, out_hbm, acc,
          eb0, eb1, xb0, xb1, sem0, sem1):
        wid = lax.axis_index("s") * 2 + lax.axis_index("c")
        lane = lax.iota(jnp.int32, 16)
        idx33 = lane * SPN
        ebufs, xbufs, sems = (eb0, eb1), (xb0, xb1), (sem0, sem1)

        def issue(c, bi, xoff):
            pltpu.async_copy(edges_hbm.at[pl.ds(c * EW, EW)], ebufs[bi],
                             sems[bi])
            pltpu.async_copy(x_hbm.at[pl.ds(xoff + c * XC, XC)],
                             xbufs[bi], sems[bi])

        def drain(bi):
            pltpu.make_async_copy(edges_hbm.at[pl.ds(0, EW)], ebufs[bi],
                                  sems[bi]).wait()
            pltpu.make_async_copy(x_hbm.at[pl.ds(0, XC)], xbufs[bi],
                                  sems[bi]).wait()

        for col in range(COLS_PER_W):
            b = wid * COLS_PER_W + col
            pltpu.sync_copy(bias_hbm, acc)
            xoff = b * N
            issue(0, 0, xoff)
            issue(1, 1, xoff)

            def pair_body(j, _):
                for bi in range(2):
                    cc = j * 2 + bi
                    drain(bi)
                    ebuf, xbuf = ebufs[bi], xbufs[bi]

                    @plsc.parallel_loop(0, GROUPS, unroll=4)
                    def group_body(g):
                        xg = xbuf[pl.ds(g * 16, 16)]
                        gidx = idx33 + g * (16 * SPN)

                        for s in range(SPN):
                            w = plsc.load_gather(ebuf, [gidx + s])
                            r = w & 0xFFFF
                            v = plsc.bitcast(w & jnp.int32(-65536),
                                             jnp.float32)
                            plsc.addupdate_scatter(acc, [r], v * xg)
                    nc = cc + 2

                    @pl.when(nc < NCHUNK)
                    def _():
                        issue(nc, bi, xoff)
                return 0

            lax.fori_loop(0, NCHUNK // 2, pair_body, 0)
            pltpu.sync_copy(acc, out_hbm.at[pl.ds(b * N, N)])

    return k(x_flat, edges, bias_flat)


def kernel(x, indices, values, bias):
    vbits = lax.bitcast_convert_type(values.astype(jnp.bfloat16),
                                     jnp.uint16).astype(jnp.int32)
    edges = (indices[0] & 0xFFFF) | lax.shift_left(vbits, 16)
    out_flat = _sc_spmm(x.reshape(-1), edges, bias.reshape(-1))
    return out_flat.reshape(B, N)


# speedup vs baseline: 3.8056x; 1.6504x over previous
import functools

import jax
import jax.numpy as jnp
from jax import lax
from jax.experimental import pallas as pl
from jax.experimental.pallas import tpu as pltpu
from jax.experimental.pallas import tpu_sc as plsc

N = 65536
B = 64
SPN = 33
XC = 512
EW = SPN * XC
GROUPS = XC // 16
NCHUNK = N // XC
NW = 32
COLS_PER_W = B // NW


def _sc_spmm(x_flat, edges, bias_flat):
    mesh = plsc.VectorSubcoreMesh(core_axis_name="c", subcore_axis_name="s")

    @functools.partial(
        pl.kernel,
        out_type=jax.ShapeDtypeStruct((B * N,), jnp.float32),
        mesh=mesh,
        scratch_types=[
            pltpu.VMEM((N,), jnp.float32),
            pltpu.VMEM((EW,), jnp.int32),
            pltpu.VMEM((EW,), jnp.int32),
            pltpu.VMEM((XC,), jnp.float32),
            pltpu.VMEM((XC,), jnp.float32),
            pltpu.SemaphoreType.DMA,
            pltpu.SemaphoreType.DMA,
        ],
        compiler_params=pltpu.CompilerParams(needs_layout_passes=False),
    )
    def k(x_hbm, edges_hbm, bias_hbm, out_hbm, acc,
          eb0, eb1, xb0, xb1, sem0, sem1):
        wid = lax.axis_index("s") * 2 + lax.axis_index("c")
        lane = lax.iota(jnp.int32, 16)
        idx33 = lane * SPN
        ebufs, xbufs, sems = (eb0, eb1), (xb0, xb1), (sem0, sem1)

        def issue(c, bi, xoff):
            pltpu.async_copy(edges_hbm.at[pl.ds(c * EW, EW)], ebufs[bi],
                             sems[bi])
            pltpu.async_copy(x_hbm.at[pl.ds(xoff + c * XC, XC)],
                             xbufs[bi], sems[bi])

        def drain(bi):
            pltpu.make_async_copy(edges_hbm.at[pl.ds(0, EW)], ebufs[bi],
                                  sems[bi]).wait()
            pltpu.make_async_copy(x_hbm.at[pl.ds(0, XC)], xbufs[bi],
                                  sems[bi]).wait()

        for col in range(COLS_PER_W):
            b = wid * COLS_PER_W + col
            pltpu.sync_copy(bias_hbm, acc)
            xoff = b * N
            issue(0, 0, xoff)
            issue(1, 1, xoff)

            def pair_body(j, _):
                for bi in range(2):
                    cc = j * 2 + bi
                    drain(bi)
                    ebuf, xbuf = ebufs[bi], xbufs[bi]

                    @plsc.parallel_loop(0, GROUPS, unroll=4)
                    def group_body(g):
                        xg = xbuf[pl.ds(g * 16, 16)]
                        g16 = g * 16

                        for s in range(SPN):
                            w = ebuf[pl.ds(s * XC + g16, 16)]
                            r = w & 0xFFFF
                            v = plsc.bitcast(w & jnp.int32(-65536),
                                             jnp.float32)
                            plsc.addupdate_scatter(acc, [r], v * xg)
                    nc = cc + 2

                    @pl.when(nc < NCHUNK)
                    def _():
                        issue(nc, bi, xoff)
                return 0

            lax.fori_loop(0, NCHUNK // 2, pair_body, 0)
            pltpu.sync_copy(acc, out_hbm.at[pl.ds(b * N, N)])

    return k(x_flat, edges, bias_flat)


def kernel(x, indices, values, bias):
    vbits = lax.bitcast_convert_type(values.astype(jnp.bfloat16),
                                     jnp.uint16).astype(jnp.int32)
    edges = (indices[0] & 0xFFFF) | lax.shift_left(vbits, 16)
    edges_b = edges.reshape(NCHUNK, XC, SPN).transpose(0, 2, 1).reshape(-1)
    out_flat = _sc_spmm(x.reshape(-1), edges_b, bias.reshape(-1))
    return out_flat.reshape(B, N)
